# baseline (speedup 1.0000x reference)
import functools

import jax
import jax.numpy as jnp
from jax import lax
from jax.experimental import pallas as pl
from jax.experimental.pallas import tpu as pltpu
from jax.experimental.pallas import tpu_sc as plsc

VOCAB_SIZE = 1000000
EMBED_DIM = 64
BATCH = 4096
LENGTH = 50

_INFO = plsc.get_sparse_core_info()
NC = _INFO.num_cores
NS = _INFO.num_subcores
NW = NC * NS
B_TOTAL = BATCH * LENGTH
CHUNK = 128
CHUNKS_TOTAL = B_TOTAL // CHUNK
CPW = CHUNKS_TOTAL // NW
BPW = CPW * CHUNK


def _sc_gather(idx_hbm, table_hbm):
    mesh = plsc.VectorSubcoreMesh(core_axis_name="c", subcore_axis_name="s")

    @functools.partial(
        pl.kernel,
        out_type=jax.ShapeDtypeStruct((B_TOTAL, EMBED_DIM), jnp.float32),
        mesh=mesh,
        scratch_types=[
            pltpu.VMEM((BPW,), jnp.int32),
            pltpu.VMEM((CHUNK, EMBED_DIM), jnp.float32),
            pltpu.VMEM((CHUNK, EMBED_DIM), jnp.float32),
            pltpu.SemaphoreType.DMA,
            pltpu.SemaphoreType.DMA,
        ],
        compiler_params=pltpu.CompilerParams(use_tc_tiling_on_sc=False),
    )
    def k(idx_ref, table_ref, out_ref, idx_v, rows_a, rows_b, gsem_a, gsem_b):
        wid = lax.axis_index("s") * NC + lax.axis_index("c")
        base = wid * BPW
        pltpu.sync_copy(idx_ref.at[pl.ds(base, BPW)], idx_v)

        def step(j, _):
            chunk_idx = idx_v.at[pl.ds(j * CHUNK, CHUNK)]
            pltpu.async_copy(table_ref.at[chunk_idx], rows_a, gsem_a).wait()
            pltpu.sync_copy(rows_a, out_ref.at[pl.ds(base + j * CHUNK, CHUNK)])
            return 0

        lax.fori_loop(0, CPW, step, 0)

    return k(idx_hbm, table_hbm)


def kernel(src_sents, weight):
    idx = src_sents.astype(jnp.int32).reshape(B_TOTAL)
    out = _sc_gather(idx, weight)
    return out.reshape(BATCH, LENGTH, EMBED_DIM)


# speedup vs baseline: 1.0423x; 1.0423x over previous
import functools

import jax
import jax.numpy as jnp
from jax import lax
from jax.experimental import pallas as pl
from jax.experimental.pallas import tpu as pltpu
from jax.experimental.pallas import tpu_sc as plsc

VOCAB_SIZE = 1000000
EMBED_DIM = 64
BATCH = 4096
LENGTH = 50

_INFO = plsc.get_sparse_core_info()
NC = _INFO.num_cores
NS = _INFO.num_subcores
NW = NC * NS
B_TOTAL = BATCH * LENGTH
CHUNK = 128
CHUNKS_TOTAL = B_TOTAL // CHUNK
CPW = CHUNKS_TOTAL // NW
BPW = CPW * CHUNK


def _sc_gather(idx_hbm, table_hbm):
    mesh = plsc.VectorSubcoreMesh(core_axis_name="c", subcore_axis_name="s")

    nbuf = 5

    @functools.partial(
        pl.kernel,
        out_type=jax.ShapeDtypeStruct((B_TOTAL, EMBED_DIM), jnp.float32),
        mesh=mesh,
        scratch_types=[
            pltpu.VMEM((BPW,), jnp.int32),
            pltpu.VMEM((nbuf, CHUNK, EMBED_DIM), jnp.float32),
            [pltpu.SemaphoreType.DMA] * nbuf,
            [pltpu.SemaphoreType.DMA] * nbuf,
        ],
        compiler_params=pltpu.CompilerParams(use_tc_tiling_on_sc=False),
    )
    def k(idx_ref, table_ref, out_ref, idx_v, rows, gsems, ssems):
        wid = lax.axis_index("s") * NC + lax.axis_index("c")
        base = wid * BPW
        pltpu.sync_copy(idx_ref.at[pl.ds(base, BPW)], idx_v)

        def gather(c, b):
            chunk_idx = idx_v.at[pl.ds(c * CHUNK, CHUNK)]
            pltpu.async_copy(table_ref.at[chunk_idx], rows.at[b], gsems[b])

        def out_slice(c):
            return out_ref.at[pl.ds(base + c * CHUNK, CHUNK)]

        def wait_gather(b):
            pltpu.make_async_copy(
                table_ref.at[pl.ds(0, CHUNK)], rows.at[b], gsems[b]
            ).wait()

        def wait_scatter(b):
            pltpu.make_async_copy(rows.at[b], out_slice(0), ssems[b]).wait()

        for b in range(nbuf):
            gather(b, b)

        @pl.loop(0, CPW - nbuf, step=nbuf)
        def pipelined(j):
            for b in range(nbuf):
                wait_gather(b)
                pltpu.async_copy(rows.at[b], out_slice(j + b), ssems[b])
            for b in range(nbuf):
                wait_scatter(b)
                gather(j + nbuf + b, b)

        for b in range(nbuf):
            wait_gather(b)
            pltpu.async_copy(rows.at[b], out_slice(CPW - nbuf + b), ssems[b])
        for b in range(nbuf):
            wait_scatter(b)

    return k(idx_hbm, table_hbm)


def kernel(src_sents, weight):
    idx = src_sents.astype(jnp.int32).reshape(B_TOTAL)
    out = _sc_gather(idx, weight)
    return out.reshape(BATCH, LENGTH, EMBED_DIM)


# speedup vs baseline: 1.0594x; 1.0165x over previous
import functools

import jax
import jax.numpy as jnp
from jax import lax
from jax.experimental import pallas as pl
from jax.experimental.pallas import tpu as pltpu
from jax.experimental.pallas import tpu_sc as plsc

VOCAB_SIZE = 1000000
EMBED_DIM = 64
BATCH = 4096
LENGTH = 50

_INFO = plsc.get_sparse_core_info()
NC = _INFO.num_cores
NS = _INFO.num_subcores
NW = NC * NS
B_TOTAL = BATCH * LENGTH
CHUNK = 128
CHUNKS_TOTAL = B_TOTAL // CHUNK
CPW = CHUNKS_TOTAL // NW
BPW = CPW * CHUNK


def _sc_gather(idx_hbm, table_hbm):
    mesh = plsc.VectorSubcoreMesh(core_axis_name="c", subcore_axis_name="s")

    nbuf = 5

    @functools.partial(
        pl.kernel,
        out_type=jax.ShapeDtypeStruct((B_TOTAL, EMBED_DIM), jnp.float32),
        mesh=mesh,
        scratch_types=[
            pltpu.VMEM((BPW,), jnp.int32),
            pltpu.VMEM((nbuf, CHUNK, EMBED_DIM), jnp.float32),
            [pltpu.SemaphoreType.DMA] * nbuf,
            [pltpu.SemaphoreType.DMA] * nbuf,
        ],
        compiler_params=pltpu.CompilerParams(use_tc_tiling_on_sc=False),
    )
    def k(idx_ref, table_ref, out_ref, idx_v, rows, gsems, ssems):
        wid = lax.axis_index("s") * NC + lax.axis_index("c")
        base = wid * BPW
        pltpu.sync_copy(idx_ref.at[pl.ds(base, BPW)], idx_v)

        def gather(c, b):
            chunk_idx = idx_v.at[pl.ds(c * CHUNK, CHUNK)]
            pltpu.async_copy(table_ref.at[chunk_idx], rows.at[b], gsems[b])

        def out_slice(c):
            return out_ref.at[pl.ds(base + c * CHUNK, CHUNK)]

        def wait_gather(b):
            pltpu.make_async_copy(
                table_ref.at[pl.ds(0, CHUNK)], rows.at[b], gsems[b]
            ).wait()

        def wait_scatter(b):
            pltpu.make_async_copy(rows.at[b], out_slice(0), ssems[b]).wait()

        for b in range(nbuf):
            gather(b, b)

        @pl.loop(0, CPW - nbuf, step=nbuf)
        def pipelined(j):
            for b in range(nbuf):
                wait_gather(b)
                pltpu.async_copy(rows.at[b], out_slice(j + b), ssems[b])
            for b in range(nbuf):
                wait_scatter(b)
                gather(j + nbuf + b, b)

        for b in range(nbuf):
            wait_gather(b)
            pltpu.async_copy(rows.at[b], out_slice(CPW - nbuf + b), ssems[b])
        for b in range(nbuf):
            wait_scatter(b)

    return k(idx_hbm, table_hbm)


def kernel(src_sents, weight):
    idx = src_sents.astype(jnp.int32).T.reshape(B_TOTAL)
    out = _sc_gather(idx, weight)
    return out.reshape(LENGTH, BATCH, EMBED_DIM).transpose(1, 0, 2)
